# baseline (speedup 1.0000x reference)
import functools

import jax
import jax.numpy as jnp
from jax import lax
from jax.experimental import pallas as pl
from jax.experimental.pallas import tpu as pltpu
from jax.experimental.pallas import tpu_sc as plsc

N = 100000
D = 128
NW = 32
C = 200
NCHUNK = N // C
ITERS = (NCHUNK + NW - 1) // NW
FULL = NCHUNK - (ITERS - 1) * NW
KB = 2

_mesh = plsc.VectorSubcoreMesh(core_axis_name="c", subcore_axis_name="s")


@functools.partial(
    pl.kernel,
    out_type=jax.ShapeDtypeStruct((N, 2 * D), jnp.float32),
    mesh=_mesh,
    scratch_types=(
        [pltpu.VMEM((C,), jnp.int32)]
        + [pltpu.VMEM((C, 2 * D), jnp.float32)] * KB
        + [pltpu.VMEM((C, D), jnp.float32)]
        + [pltpu.VMEM((16,), jnp.int32)]
        + [pltpu.SemaphoreType.DMA] * (1 + 3 * KB)
    ),
)
def _sc_gather_concat(ctx_hbm, obj_hbm, idx_hbm, ev_hbm, out_hbm, *scr):
    idx_v = scr[0]
    comb = scr[1:1 + KB]
    ctx_tmp = scr[1 + KB]
    ev_v = scr[2 + KB]
    sem_misc = scr[3 + KB]
    sem_ci = scr[4 + KB:4 + 2 * KB]
    sem_oi = scr[4 + 2 * KB:4 + 3 * KB]
    sem_w = scr[4 + 3 * KB:4 + 4 * KB]

    wid = lax.axis_index("s") * 2 + lax.axis_index("c")
    last_ok = wid < FULL

    pltpu.sync_copy(ev_hbm, ev_v)
    shuffled = ev_v[...][0] != 0

    def rows(i):
        return pl.ds((wid + i * NW) * C, C)

    def guarded(i, fn):
        if i == ITERS - 1:
            pl.when(last_ok)(fn)
        else:
            fn()

    @pl.when(jnp.logical_not(shuffled))
    def _identity_path():
        in_d = [None] * ITERS
        out_d = [None] * ITERS

        def start_in(i):
            b = i % KB
            ci = pltpu.make_async_copy(
                ctx_hbm.at[rows(i)], comb[b].at[:, pl.ds(0, D)], sem_ci[b])
            oi = pltpu.make_async_copy(
                obj_hbm.at[rows(i)], comb[b].at[:, pl.ds(D, D)], sem_oi[b])
            in_d[i] = (ci, oi)
            guarded(i, ci.start)
            guarded(i, oi.start)

        def start_out(i):
            b = i % KB
            w = pltpu.make_async_copy(comb[b], out_hbm.at[rows(i)], sem_w[b])
            out_d[i] = w
            guarded(i, in_d[i][0].wait)
            guarded(i, in_d[i][1].wait)
            guarded(i, w.start)

        for i in range(ITERS):
            if i >= KB:
                guarded(i - KB, out_d[i - KB].wait)
            start_in(i)
            if i >= 1:
                start_out(i - 1)
        start_out(ITERS - 1)
        for j in range(ITERS - KB, ITERS):
            guarded(j, out_d[j].wait)

    @pl.when(shuffled)
    def _gather_path():
        for i in range(ITERS):
            def body(i=i):
                pltpu.sync_copy(idx_hbm.at[rows(i)], idx_v)
                pltpu.async_copy(ctx_hbm.at[idx_v], ctx_tmp, sem_misc).wait()
                pltpu.sync_copy(ctx_tmp, out_hbm.at[rows(i), pl.ds(0, D)])
                pltpu.sync_copy(obj_hbm.at[rows(i)], ctx_tmp)
                pltpu.sync_copy(ctx_tmp, out_hbm.at[rows(i), pl.ds(D, D)])
            guarded(i, body)


def kernel(context_output, object_output, eval_random):
    num = context_output.shape[0]
    with jax.ensure_compile_time_eval():
        perm_idx = jnp.asarray(
            jax.random.permutation(jax.random.key(42), num), jnp.int32)
        identity_idx = jnp.arange(num, dtype=jnp.int32)
    random_idx = jnp.where(eval_random, perm_idx, identity_idx)
    ev = jnp.broadcast_to(jnp.asarray(eval_random, jnp.int32), (16,))
    return _sc_gather_concat(context_output, object_output, random_idx, ev)


# speedup vs baseline: 1.0347x; 1.0347x over previous
import functools

import jax
import jax.numpy as jnp
from jax import lax
from jax.experimental import pallas as pl
from jax.experimental.pallas import tpu as pltpu
from jax.experimental.pallas import tpu_sc as plsc

N = 100000
D = 128
NW = 32
C = 200
NCHUNK = N // C
ITERS = (NCHUNK + NW - 1) // NW
FULL = NCHUNK - (ITERS - 1) * NW
KC = 3
KO = 2
TILE_OBJ = tuple(i for i in range(ITERS) if i % 4 == 2)

_mesh = plsc.VectorSubcoreMesh(core_axis_name="c", subcore_axis_name="s")


@functools.partial(
    pl.kernel,
    out_type=jax.ShapeDtypeStruct((N, 2 * D), jnp.float32),
    mesh=_mesh,
    scratch_types=(
        [pltpu.VMEM((C,), jnp.int32)] * KC
        + [pltpu.VMEM((C, D), jnp.float32)] * KC
        + [pltpu.VMEM_SHARED((16, KO, C, D), jnp.float32)]
        + [pltpu.VMEM((16,), jnp.int32)]
        + [pltpu.SemaphoreType.DMA] * (1 + 2 * KC + 2 * KO)
    ),
)
def _sc_gather_concat(ctx_hbm, obj_hbm, idx_hbm, ev_hbm, out_hbm, *scr):
    idx_bufs = scr[:KC]
    p = KC
    ctx_v = scr[p:p + KC]; p += KC
    obj_s = scr[p]; p += 1
    ev_v = scr[p]; p += 1
    sem_idx = scr[p]; p += 1
    sem_g = scr[p:p + KC]; p += KC
    sem_wg = scr[p:p + KC]; p += KC
    sem_o = scr[p:p + KO]; p += KO
    sem_wo = scr[p:p + KO]; p += KO
    sid = lax.axis_index("s")

    wid = lax.axis_index("s") * 2 + lax.axis_index("c")
    last_ok = wid < FULL

    pltpu.sync_copy(ev_hbm, ev_v)
    shuffled = ev_v[...][0] != 0

    def rows(i):
        return pl.ds((wid + i * NW) * C, C)

    def guarded(i, fn):
        if i == ITERS - 1:
            pl.when(last_ok)(fn)
        else:
            fn()

    @pl.when(jnp.logical_not(shuffled))
    def _identity_path():
        jobs_a = []
        for i in range(ITERS):
            jobs_a.append((i, 0))
            if i in TILE_OBJ:
                jobs_a.append((i, 1))
        jobs_b = [i for i in range(ITERS) if i not in TILE_OBJ]
        na, nb = len(jobs_a), len(jobs_b)
        a_in = [None] * na
        a_out = [None] * na
        b_in = [None] * nb
        b_out = [None] * nb

        def a_start(k):
            i, h = jobs_a[k]
            b = k % KC
            src = (ctx_hbm if h == 0 else obj_hbm).at[rows(i)]
            a_in[k] = pltpu.make_async_copy(src, ctx_v[b], sem_g[b])
            guarded(i, a_in[k].start)

        def a_finish(k):
            i, h = jobs_a[k]
            b = k % KC
            a_out[k] = pltpu.make_async_copy(
                ctx_v[b], out_hbm.at[rows(i), pl.ds(h * D, D)], sem_wg[b])
            guarded(i, a_in[k].wait)
            guarded(i, a_out[k].start)

        def b_start(k):
            i = jobs_b[k]
            b = k % KO
            b_in[k] = pltpu.make_async_copy(
                obj_hbm.at[rows(i)], obj_s.at[sid, b], sem_o[b])
            guarded(i, b_in[k].start)

        def b_finish(k):
            i = jobs_b[k]
            b = k % KO
            b_out[k] = pltpu.make_async_copy(
                obj_s.at[sid, b], out_hbm.at[rows(i), pl.ds(D, D)],
                sem_wo[b])
            guarded(i, b_in[k].wait)
            guarded(i, b_out[k].start)

        for step in range(max(na, nb)):
            if step < na:
                if step >= KC:
                    guarded(jobs_a[step - KC][0], a_out[step - KC].wait)
                a_start(step)
                if step >= 1:
                    a_finish(step - 1)
            if step < nb:
                if step >= KO:
                    guarded(jobs_b[step - KO], b_out[step - KO].wait)
                b_start(step)
                if step >= 1:
                    b_finish(step - 1)
        a_finish(na - 1)
        b_finish(nb - 1)
        for k in range(na - KC, na):
            guarded(jobs_a[k][0], a_out[k].wait)
        for k in range(nb - KO, nb):
            guarded(jobs_b[k], b_out[k].wait)

    @pl.when(shuffled)
    def _gather_path():
        in_d = [None] * ITERS
        out_d = [None] * ITERS

        def start_in(i):
            b = i % KC
            bo = i % KO
            idd = pltpu.make_async_copy(
                idx_hbm.at[rows(i)], idx_bufs[b], sem_idx)
            guarded(i, idd.start)
            guarded(i, idd.wait)
            g = pltpu.make_async_copy(
                ctx_hbm.at[idx_bufs[b]], ctx_v[b], sem_g[b])
            o = pltpu.make_async_copy(
                obj_hbm.at[rows(i)], obj_s.at[sid, bo], sem_o[bo])
            in_d[i] = (g, o)
            guarded(i, g.start)
            guarded(i, o.start)

        def start_out(i):
            b = i % KC
            bo = i % KO
            wg = pltpu.make_async_copy(
                ctx_v[b], out_hbm.at[rows(i), pl.ds(0, D)], sem_wg[b])
            wo = pltpu.make_async_copy(
                obj_s.at[sid, bo], out_hbm.at[rows(i), pl.ds(D, D)],
                sem_wo[bo])
            out_d[i] = (wg, wo)
            guarded(i, in_d[i][0].wait)
            guarded(i, wg.start)
            guarded(i, in_d[i][1].wait)
            guarded(i, wo.start)

        for i in range(ITERS):
            if i >= KO:
                guarded(i - KO, out_d[i - KO][0].wait)
                guarded(i - KO, out_d[i - KO][1].wait)
            start_in(i)
            if i >= 1:
                start_out(i - 1)
        start_out(ITERS - 1)
        for j in range(ITERS - KO, ITERS):
            guarded(j, out_d[j][0].wait)
            guarded(j, out_d[j][1].wait)


def kernel(context_output, object_output, eval_random):
    num = context_output.shape[0]
    with jax.ensure_compile_time_eval():
        perm_idx = jnp.asarray(
            jax.random.permutation(jax.random.key(42), num), jnp.int32)
        identity_idx = jnp.arange(num, dtype=jnp.int32)
    random_idx = jnp.where(eval_random, perm_idx, identity_idx)
    ev = jnp.broadcast_to(jnp.asarray(eval_random, jnp.int32), (16,))
    return _sc_gather_concat(context_output, object_output, random_idx, ev)


# speedup vs baseline: 1.0362x; 1.0015x over previous
import functools

import jax
import jax.numpy as jnp
from jax import lax
from jax.experimental import pallas as pl
from jax.experimental.pallas import tpu as pltpu
from jax.experimental.pallas import tpu_sc as plsc

N = 100000
D = 128
NW = 32
C = 200
NCHUNK = N // C
ITERS = (NCHUNK + NW - 1) // NW
FULL = NCHUNK - (ITERS - 1) * NW
KC = 3
KO = 2

_mesh = plsc.VectorSubcoreMesh(core_axis_name="c", subcore_axis_name="s")


@functools.partial(
    pl.kernel,
    out_type=jax.ShapeDtypeStruct((N, 2 * D), jnp.float32),
    mesh=_mesh,
    scratch_types=(
        [pltpu.VMEM((C,), jnp.int32)] * KC
        + [pltpu.VMEM((C, D), jnp.float32)] * KC
        + [pltpu.VMEM_SHARED((16, KO, C, D), jnp.float32)]
        + [pltpu.VMEM((16,), jnp.int32)]
        + [pltpu.SemaphoreType.DMA] * (1 + 2 * KC + 2 * KO)
    ),
)
def _sc_gather_concat(ctx_hbm, obj_hbm, idx_hbm, ev_hbm, out_hbm, *scr):
    idx_bufs = scr[:KC]
    p = KC
    ctx_v = scr[p:p + KC]; p += KC
    obj_s = scr[p]; p += 1
    ev_v = scr[p]; p += 1
    sem_idx = scr[p]; p += 1
    sem_g = scr[p:p + KC]; p += KC
    sem_wg = scr[p:p + KC]; p += KC
    sem_o = scr[p:p + KO]; p += KO
    sem_wo = scr[p:p + KO]; p += KO
    sid = lax.axis_index("s")

    wid = lax.axis_index("s") * 2 + lax.axis_index("c")
    last_ok = wid < FULL

    pltpu.sync_copy(ev_hbm, ev_v)
    shuffled = ev_v[...][0] != 0

    def rows(i):
        return pl.ds((wid + i * NW) * C, C)

    def guarded(i, fn):
        if i == ITERS - 1:
            pl.when(last_ok)(fn)
        else:
            fn()

    def pipeline(make_ctx_in):
        ctx_in = [None] * ITERS
        ctx_out = [None] * ITERS
        obj_in = [None] * ITERS
        obj_out = [None] * ITERS

        def finish(j):
            bc, bo = j % KC, j % KO
            ctx_out[j] = pltpu.make_async_copy(
                ctx_v[bc], out_hbm.at[rows(j), pl.ds(0, D)], sem_wg[bc])
            obj_out[j] = pltpu.make_async_copy(
                obj_s.at[sid, bo], out_hbm.at[rows(j), pl.ds(D, D)],
                sem_wo[bo])
            guarded(j, ctx_in[j].wait)
            guarded(j, ctx_out[j].start)
            guarded(j, obj_in[j].wait)
            guarded(j, obj_out[j].start)

        for i in range(ITERS):
            if i >= KC:
                guarded(i - KC, ctx_out[i - KC].wait)
            if i >= KO:
                guarded(i - KO, obj_out[i - KO].wait)
            bc, bo = i % KC, i % KO
            ctx_in[i] = make_ctx_in(i, ctx_v[bc], sem_g[bc])
            obj_in[i] = pltpu.make_async_copy(
                obj_hbm.at[rows(i)], obj_s.at[sid, bo], sem_o[bo])
            guarded(i, ctx_in[i].start)
            guarded(i, obj_in[i].start)
            if i >= 1:
                finish(i - 1)
        finish(ITERS - 1)
        for j in range(max(0, ITERS - KC), ITERS):
            guarded(j, ctx_out[j].wait)
        for j in range(max(0, ITERS - KO), ITERS):
            guarded(j, obj_out[j].wait)

    @pl.when(jnp.logical_not(shuffled))
    def _identity_path():
        pipeline(lambda i, dst, sem: pltpu.make_async_copy(
            ctx_hbm.at[rows(i)], dst, sem))

    @pl.when(shuffled)
    def _gather_path():
        def gather_in(i, dst, sem):
            b = i % KC
            idd = pltpu.make_async_copy(
                idx_hbm.at[rows(i)], idx_bufs[b], sem_idx)
            guarded(i, idd.start)
            guarded(i, idd.wait)
            return pltpu.make_async_copy(ctx_hbm.at[idx_bufs[b]], dst, sem)

        pipeline(gather_in)


def kernel(context_output, object_output, eval_random):
    num = context_output.shape[0]
    with jax.ensure_compile_time_eval():
        perm_idx = jnp.asarray(
            jax.random.permutation(jax.random.key(42), num), jnp.int32)
        identity_idx = jnp.arange(num, dtype=jnp.int32)
    random_idx = jnp.where(eval_random, perm_idx, identity_idx)
    ev = jnp.broadcast_to(jnp.asarray(eval_random, jnp.int32), (16,))
    return _sc_gather_concat(context_output, object_output, random_idx, ev)
